# baseline (speedup 1.0000x reference)
import functools

import jax
import jax.numpy as jnp
from jax import lax
from jax.experimental import pallas as pl
from jax.experimental.pallas import tpu as pltpu
from jax.experimental.pallas import tpu_sc as plsc

VOCAB = 38
SEQ = 64
EMB = 128
HID = 256
BATCH = 4096

NPAIR = SEQ // 2
VPAD = 40
PROWS = VPAD * VPAD
NROWS = NPAIR * PROWS
HIDW = HID // 2
QBIAS = 2048
L = 16
NC = 2
NS = 16
NW = NC * NS
BPW = BATCH // NW


SB = 8


def _m_body(emb_ref, w1_ref, b1_ref, m_ref, mx_ref):
    first = (pl.program_id(0) == 0).astype(jnp.float32)
    mxv = jnp.float32(0.0)
    for k in range(SB):
        m = jnp.dot(emb_ref[...], w1_ref[k],
                    preferred_element_type=jnp.float32)
        if k == 0:
            m = m + b1_ref[...] * first
        m_ref[k] = m
        mxv = jnp.maximum(mxv, jnp.max(jnp.abs(m)))
    mx_ref[0] = jnp.full((8, EMB), mxv, jnp.float32)


def _build_m(emb_pad, w1r, b1):
    return pl.pallas_call(
        _m_body,
        grid=(SEQ // SB,),
        in_specs=[
            pl.BlockSpec((VPAD, EMB), lambda s: (0, 0)),
            pl.BlockSpec((SB, EMB, HID), lambda s: (s, 0, 0)),
            pl.BlockSpec((1, HID), lambda s: (0, 0)),
        ],
        out_specs=[
            pl.BlockSpec((SB, VPAD, HID), lambda s: (s, 0, 0)),
            pl.BlockSpec((1, 8, EMB), lambda s: (s, 0, 0)),
        ],
        out_shape=[
            jax.ShapeDtypeStruct((SEQ, VPAD, HID), jnp.float32),
            jax.ShapeDtypeStruct((SEQ // SB, 8, EMB), jnp.float32),
        ],
    )(emb_pad, w1r, b1.reshape(1, HID))


def _pack_body(ma_ref, mb_ref, inv_ref, out_ref):
    def quant_pack(m):
        q = jnp.clip(m * inv_ref[0, 0] + (QBIAS // 2 + 0.5), 1.0, 2047.0)
        qi = q.astype(jnp.int32)
        return qi[:, :HIDW] + qi[:, HIDW:] * 65536

    pa = quant_pack(ma_ref[0])
    pb = quant_pack(mb_ref[0])
    out_ref[0] = pa[:, None, :] + pb[None, :, :]


def _build_table(m, inv_delta):
    return pl.pallas_call(
        _pack_body,
        grid=(NPAIR,),
        in_specs=[
            pl.BlockSpec((1, VPAD, HID), lambda s: (s, 0, 0)),
            pl.BlockSpec((1, VPAD, HID), lambda s: (s + NPAIR, 0, 0)),
            pl.BlockSpec((1, 1), lambda s: (0, 0)),
        ],
        out_specs=pl.BlockSpec((1, VPAD, VPAD, HIDW), lambda s: (s, 0, 0, 0)),
        out_shape=jax.ShapeDtypeStruct((NPAIR, VPAD, VPAD, HIDW), jnp.int32),
    )(m, m, inv_delta.reshape(1, 1))


@functools.cache
def _make_sc_forward():
    mesh = plsc.VectorSubcoreMesh(core_axis_name="c", subcore_axis_name="s")
    return functools.partial(
        pl.kernel,
        mesh=mesh,
        out_type=jax.ShapeDtypeStruct((BATCH,), jnp.float32),
        scratch_types=[
            pltpu.VMEM((BPW * SEQ,), jnp.int32),
            pltpu.VMEM((4, 4 * NPAIR), jnp.int32),
            pltpu.VMEM((4, 4 * NPAIR, HIDW), jnp.int32),
            pltpu.VMEM((HID,), jnp.float32),
            pltpu.VMEM((L,), jnp.float32),
            pltpu.VMEM((BPW,), jnp.float32),
            pltpu.SemaphoreType.DMA,
            pltpu.SemaphoreType.DMA,
            pltpu.SemaphoreType.DMA,
            pltpu.SemaphoreType.DMA,
        ],
    )(_sc_body)


_GDN = lax.GatherDimensionNumbers(
    offset_dims=(), collapsed_slice_dims=(0,), start_index_map=(0,))


def _lane_perm(v, idx):
    return lax.gather(v, idx[:, None], _GDN, slice_sizes=(1,),
                      mode=lax.GatherScatterMode.PROMISE_IN_BOUNDS)


def _lane_sum(v):
    lanes = lax.iota(jnp.int32, L)
    for k in (1, 2, 4, 8):
        v = v + _lane_perm(v, lanes ^ k)
    return v


def _sc_body(x_hbm, t2_hbm, w2_hbm, b2_hbm, out_hbm,
             xs_v, idx_v, rows_v, w2_v, b2_v, log_v,
             sem0, sem1, sem2, sem3):
    wid = lax.axis_index("s") * NC + lax.axis_index("c")
    base = wid * BPW

    pltpu.sync_copy(x_hbm.at[pl.ds(base * SEQ, BPW * SEQ)], xs_v)
    pltpu.sync_copy(w2_hbm, w2_v)
    pltpu.sync_copy(b2_hbm, b2_v)

    sems = (sem0, sem1, sem2, sem3)
    offs = [(lax.iota(jnp.int32, L) + k * L) * PROWS for k in range(NPAIR // L)]

    def compute_idx(i, b):
        for q in range(4):
            iq = i + q
            xv = [xs_v[pl.ds(iq * SEQ + k * L, L)] for k in range(SEQ // L)]
            for k in range(NPAIR // L):
                idx_v[b, pl.ds(q * NPAIR + k * L, L)] = (
                    offs[k] + xv[k] * VPAD + xv[k + NPAIR // L])

    def start(b):
        pltpu.async_copy(t2_hbm.at[idx_v.at[b]], rows_v.at[b], sems[b])

    def wait(b):
        pltpu.make_async_copy(t2_hbm.at[idx_v.at[b]], rows_v.at[b],
                              sems[b]).wait()

    lanes = lax.iota(jnp.int32, L)
    bias_c = jnp.float32(NPAIR * QBIAS)

    def one_chunk(q, b, c, dot):
        co = c * L
        s_lo = None
        s_hi = None
        for g in range(4):
            w = rows_v[b, q * NPAIR + 8 * g, pl.ds(co, L)]
            for j in range(1, 8):
                w = w + rows_v[b, q * NPAIR + 8 * g + j, pl.ds(co, L)]
            lo = lax.bitwise_and(w, 65535)
            hi = lax.shift_right_logical(w, 16)
            s_lo = lo if g == 0 else s_lo + lo
            s_hi = hi if g == 0 else s_hi + hi
        hl = jnp.maximum(s_lo.astype(jnp.float32) - bias_c, 0.0)
        hh = jnp.maximum(s_hi.astype(jnp.float32) - bias_c, 0.0)
        return (dot + hl * w2_v[pl.ds(co, L)]
                + hh * w2_v[pl.ds(HIDW + co, L)])

    def accumulate(lane_id, q, b, logacc):
        def chunk_body(c, dot):
            dot = one_chunk(q, b, 2 * c, dot)
            return one_chunk(q, b, 2 * c + 1, dot)

        dot = lax.fori_loop(0, HIDW // (2 * L), chunk_body, b2_v[...])
        tot = _lane_sum(dot)
        return logacc + jnp.where(lanes == lane_id, tot, 0.0)

    for r in range(3):
        compute_idx(4 * r, r)
        start(r)

    def quad_body(p, _):
        logacc = jnp.zeros((L,), jnp.float32)
        for r in range(4):
            g = 4 * p + r
            pre = (r + 3) % 4

            def prefetch():
                compute_idx(4 * (g + 3), pre)
                start(pre)

            if r == 0:
                prefetch()
            else:
                pl.when(p < BPW // 16 - 1)(prefetch)

            wait(r)
            for q in range(4):
                logacc = accumulate(4 * r + q, q, r, logacc)
        log_v[pl.ds(p * L, L)] = logacc
        return 0

    lax.fori_loop(0, BPW // 16, quad_body, 0)

    pltpu.sync_copy(log_v, out_hbm.at[pl.ds(base, BPW)])


def kernel(batch_x, emb_table, W1, b1, W2, b2):
    w1r = W1.reshape(SEQ, EMB, HID)
    emb_pad = jnp.zeros((VPAD, EMB), jnp.float32).at[:VOCAB].set(emb_table)
    m, mx = _build_m(emb_pad, w1r, b1)

    delta = jnp.maximum(jnp.max(mx), 1e-30) / 1023.0
    table = _build_table(m, 1.0 / delta).reshape(NROWS, HIDW)

    x_flat = batch_x.astype(jnp.int32).reshape(-1)
    w2 = W2.reshape(HID) * delta
    b2_pad = jnp.zeros((L,), jnp.float32).at[0].set(b2[0])
    return _make_sc_forward()(x_flat, table, w2, b2_pad)


# speedup vs baseline: 1.0383x; 1.0383x over previous
import functools

import jax
import jax.numpy as jnp
from jax import lax
from jax.experimental import pallas as pl
from jax.experimental.pallas import tpu as pltpu
from jax.experimental.pallas import tpu_sc as plsc

VOCAB = 38
SEQ = 64
EMB = 128
HID = 256
BATCH = 4096

NPAIR = SEQ // 2
VPAD = 40
PROWS = VPAD * VPAD
NROWS = NPAIR * PROWS
HIDW = HID // 2
QBIAS = 2048
L = 16
NC = 2
NS = 16
NW = NC * NS
BPW = BATCH // NW


SB = 8


def _m_body(emb_ref, w1_ref, b1_ref, m_ref, mx_ref):
    first = (pl.program_id(0) == 0).astype(jnp.float32)
    mxv = jnp.float32(0.0)
    for k in range(SB):
        m = jnp.dot(emb_ref[...], w1_ref[k],
                    preferred_element_type=jnp.float32)
        if k == 0:
            m = m + b1_ref[...] * first
        m_ref[k] = m
        mxv = jnp.maximum(mxv, jnp.max(jnp.abs(m)))
    mx_ref[0] = jnp.full((8, EMB), mxv, jnp.float32)


def _build_m(emb_pad, w1r, b1):
    return pl.pallas_call(
        _m_body,
        grid=(SEQ // SB,),
        in_specs=[
            pl.BlockSpec((VPAD, EMB), lambda s: (0, 0)),
            pl.BlockSpec((SB, EMB, HID), lambda s: (s, 0, 0)),
            pl.BlockSpec((1, HID), lambda s: (0, 0)),
        ],
        out_specs=[
            pl.BlockSpec((SB, VPAD, HID), lambda s: (s, 0, 0)),
            pl.BlockSpec((1, 8, EMB), lambda s: (s, 0, 0)),
        ],
        out_shape=[
            jax.ShapeDtypeStruct((SEQ, VPAD, HID), jnp.float32),
            jax.ShapeDtypeStruct((SEQ // SB, 8, EMB), jnp.float32),
        ],
    )(emb_pad, w1r, b1.reshape(1, HID))


def _pack_body(ma_ref, mb_ref, inv_ref, out_ref):
    def quant_pack(m):
        q = jnp.clip(m * inv_ref[0, 0] + (QBIAS // 2 + 0.5), 1.0, 2047.0)
        qi = q.astype(jnp.int32)
        return qi[:, :HIDW] + qi[:, HIDW:] * 65536

    pa = quant_pack(ma_ref[0])
    pb = quant_pack(mb_ref[0])
    out_ref[0] = pa[:, None, :] + pb[None, :, :]


def _build_table(m, inv_delta):
    return pl.pallas_call(
        _pack_body,
        grid=(NPAIR,),
        in_specs=[
            pl.BlockSpec((1, VPAD, HID), lambda s: (s, 0, 0)),
            pl.BlockSpec((1, VPAD, HID), lambda s: (s + NPAIR, 0, 0)),
            pl.BlockSpec((1, 1), lambda s: (0, 0)),
        ],
        out_specs=pl.BlockSpec((1, VPAD, VPAD, HIDW), lambda s: (s, 0, 0, 0)),
        out_shape=jax.ShapeDtypeStruct((NPAIR, VPAD, VPAD, HIDW), jnp.int32),
    )(m, m, inv_delta.reshape(1, 1))


@functools.cache
def _make_sc_forward():
    mesh = plsc.VectorSubcoreMesh(core_axis_name="c", subcore_axis_name="s")
    return functools.partial(
        pl.kernel,
        mesh=mesh,
        out_type=jax.ShapeDtypeStruct((BATCH,), jnp.float32),
        scratch_types=[
            pltpu.VMEM((BPW * SEQ,), jnp.int32),
            pltpu.VMEM((4, 4 * NPAIR), jnp.int32),
            pltpu.VMEM((4, 4 * NPAIR, HIDW), jnp.int32),
            pltpu.VMEM((HID,), jnp.float32),
            pltpu.VMEM((L,), jnp.float32),
            pltpu.VMEM((BPW,), jnp.float32),
            pltpu.SemaphoreType.DMA,
            pltpu.SemaphoreType.DMA,
            pltpu.SemaphoreType.DMA,
            pltpu.SemaphoreType.DMA,
        ],
    )(_sc_body)


_GDN = lax.GatherDimensionNumbers(
    offset_dims=(), collapsed_slice_dims=(0,), start_index_map=(0,))


def _lane_perm(v, idx):
    return lax.gather(v, idx[:, None], _GDN, slice_sizes=(1,),
                      mode=lax.GatherScatterMode.PROMISE_IN_BOUNDS)


def _lane_sum(v):
    lanes = lax.iota(jnp.int32, L)
    for k in (1, 2, 4, 8):
        v = v + _lane_perm(v, lanes ^ k)
    return v


def _sc_body(x_hbm, t2_hbm, w2_hbm, b2_hbm, out_hbm,
             xs_v, idx_v, rows_v, w2_v, b2_v, log_v,
             sem0, sem1, sem2, sem3):
    wid = lax.axis_index("s") * NC + lax.axis_index("c")
    base = wid * BPW

    pltpu.sync_copy(x_hbm.at[pl.ds(base * SEQ, BPW * SEQ)], xs_v)
    pltpu.sync_copy(w2_hbm, w2_v)
    pltpu.sync_copy(b2_hbm, b2_v)

    sems = (sem0, sem1, sem2, sem3)
    offs = [(lax.iota(jnp.int32, L) + k * L) * PROWS for k in range(NPAIR // L)]

    def compute_idx(i, b):
        for q in range(4):
            iq = i + q
            xv = [xs_v[pl.ds(iq * SEQ + k * L, L)] for k in range(SEQ // L)]
            for k in range(NPAIR // L):
                idx_v[b, pl.ds(q * NPAIR + k * L, L)] = (
                    offs[k] + xv[k] * VPAD + xv[k + NPAIR // L])

    def start(b):
        pltpu.async_copy(t2_hbm.at[idx_v.at[b]], rows_v.at[b], sems[b])

    def wait(b):
        pltpu.make_async_copy(t2_hbm.at[idx_v.at[b]], rows_v.at[b],
                              sems[b]).wait()

    lanes = lax.iota(jnp.int32, L)
    bias_c = jnp.float32(NPAIR * QBIAS)

    def one_chunk(q, b, c, dot):
        co = c * L
        s_lo = None
        s_hi = None
        for g in range(4):
            w = rows_v[b, q * NPAIR + 8 * g, pl.ds(co, L)]
            for j in range(1, 8):
                w = w + rows_v[b, q * NPAIR + 8 * g + j, pl.ds(co, L)]
            lo = lax.bitwise_and(w, 65535)
            hi = lax.shift_right_logical(w, 16)
            s_lo = lo if g == 0 else s_lo + lo
            s_hi = hi if g == 0 else s_hi + hi
        hl = jnp.maximum(s_lo.astype(jnp.float32) - bias_c, 0.0)
        hh = jnp.maximum(s_hi.astype(jnp.float32) - bias_c, 0.0)
        return (dot + hl * w2_v[pl.ds(co, L)]
                + hh * w2_v[pl.ds(HIDW + co, L)])

    def accumulate(lane_id, q, b, logacc):
        def chunk_body(c, dot):
            dot = one_chunk(q, b, 2 * c, dot)
            return one_chunk(q, b, 2 * c + 1, dot)

        dot = lax.fori_loop(0, HIDW // (2 * L), chunk_body, b2_v[...])
        tot = _lane_sum(dot)
        return logacc + jnp.where(lanes == lane_id, tot, 0.0)

    compute_idx(0, 0)
    start(0)

    def pair_body(p, logacc):
        i0 = 8 * p
        compute_idx(i0 + 4, 1)
        start(1)
        wait(0)
        for q in range(4):
            logacc = accumulate((i0 + q) & (L - 1), q, 0, logacc)

        @pl.when(p < BPW // 8 - 1)
        def _():
            compute_idx(i0 + 8, 0)
            start(0)

        wait(1)
        for q in range(4):
            logacc = accumulate((i0 + 4 + q) & (L - 1), q, 1, logacc)

        group_done = (p & 1) == 1

        @pl.when(group_done)
        def _():
            log_v[pl.ds((p >> 1) * L, L)] = logacc

        return jnp.where(group_done, 0.0, logacc)

    lax.fori_loop(0, BPW // 8, pair_body, jnp.zeros((L,), jnp.float32))

    pltpu.sync_copy(log_v, out_hbm.at[pl.ds(base, BPW)])


def kernel(batch_x, emb_table, W1, b1, W2, b2):
    w1r = W1.reshape(SEQ, EMB, HID)
    emb_pad = jnp.zeros((VPAD, EMB), jnp.float32).at[:VOCAB].set(emb_table)
    m, mx = _build_m(emb_pad, w1r, b1)

    delta = jnp.maximum(jnp.max(mx), 1e-30) / 1023.0
    table = _build_table(m, 1.0 / delta).reshape(NROWS, HIDW)

    x_flat = batch_x.astype(jnp.int32).reshape(-1)
    w2 = W2.reshape(HID) * delta
    b2_pad = jnp.zeros((L,), jnp.float32).at[0].set(b2[0])
    return _make_sc_forward()(x_flat, table, w2, b2_pad)


# speedup vs baseline: 1.1798x; 1.1363x over previous
import functools

import jax
import jax.numpy as jnp
from jax import lax
from jax.experimental import pallas as pl
from jax.experimental.pallas import tpu as pltpu
from jax.experimental.pallas import tpu_sc as plsc

VOCAB = 38
SEQ = 64
EMB = 128
HID = 256
BATCH = 4096

NPAIR = SEQ // 2
VPAD = 40
PROWS = VPAD * VPAD
NROWS = NPAIR * PROWS
HIDW = HID // 2
QBIAS = 2048
L = 16
NC = 2
NS = 16
NW = NC * NS
BPW = BATCH // NW


SB = 8


def _m_body(emb_ref, w1_ref, b1_ref, m_ref, mx_ref):
    first = (pl.program_id(0) == 0).astype(jnp.float32)
    mxv = jnp.float32(0.0)
    for k in range(SB):
        m = jnp.dot(emb_ref[...], w1_ref[k],
                    preferred_element_type=jnp.float32)
        if k == 0:
            m = m + b1_ref[...] * first
        m_ref[k] = m
        mxv = jnp.maximum(mxv, jnp.max(jnp.abs(m)))
    mx_ref[0] = jnp.full((8, EMB), mxv, jnp.float32)


def _build_m(emb_pad, w1r, b1):
    return pl.pallas_call(
        _m_body,
        grid=(SEQ // SB,),
        in_specs=[
            pl.BlockSpec((VPAD, EMB), lambda s: (0, 0)),
            pl.BlockSpec((SB, EMB, HID), lambda s: (s, 0, 0)),
            pl.BlockSpec((1, HID), lambda s: (0, 0)),
        ],
        out_specs=[
            pl.BlockSpec((SB, VPAD, HID), lambda s: (s, 0, 0)),
            pl.BlockSpec((1, 8, EMB), lambda s: (s, 0, 0)),
        ],
        out_shape=[
            jax.ShapeDtypeStruct((SEQ, VPAD, HID), jnp.float32),
            jax.ShapeDtypeStruct((SEQ // SB, 8, EMB), jnp.float32),
        ],
    )(emb_pad, w1r, b1.reshape(1, HID))


PB = 4


def _pack_body(ma_ref, mb_ref, inv_ref, out_ref):
    def quant_pack(m):
        q = jnp.clip(m * inv_ref[0, 0] + (QBIAS // 2 + 0.5), 1.0, 2047.0)
        qi = q.astype(jnp.int32)
        return qi[:, :HIDW] + qi[:, HIDW:] * 65536

    for k in range(PB):
        pa = quant_pack(ma_ref[k])
        pb = quant_pack(mb_ref[k])
        out_ref[k] = pa[:, None, :] + pb[None, :, :]


def _build_table(m, inv_delta):
    return pl.pallas_call(
        _pack_body,
        grid=(NPAIR // PB,),
        in_specs=[
            pl.BlockSpec((PB, VPAD, HID), lambda s: (s, 0, 0)),
            pl.BlockSpec((PB, VPAD, HID),
                         lambda s: (s + NPAIR // PB, 0, 0)),
            pl.BlockSpec((1, 1), lambda s: (0, 0)),
        ],
        out_specs=pl.BlockSpec((PB, VPAD, VPAD, HIDW),
                               lambda s: (s, 0, 0, 0)),
        out_shape=jax.ShapeDtypeStruct((NPAIR, VPAD, VPAD, HIDW), jnp.int32),
    )(m, m, inv_delta.reshape(1, 1))


@functools.cache
def _make_sc_forward():
    mesh = plsc.VectorSubcoreMesh(core_axis_name="c", subcore_axis_name="s")
    return functools.partial(
        pl.kernel,
        mesh=mesh,
        out_type=jax.ShapeDtypeStruct((BATCH,), jnp.float32),
        scratch_types=[
            pltpu.VMEM((BPW * SEQ,), jnp.int32),
            pltpu.VMEM((4, 4 * NPAIR), jnp.int32),
            pltpu.VMEM((4, 4 * NPAIR, HIDW), jnp.int32),
            pltpu.VMEM((HID,), jnp.float32),
            pltpu.VMEM((L,), jnp.float32),
            pltpu.VMEM((BPW,), jnp.float32),
            pltpu.SemaphoreType.DMA,
            pltpu.SemaphoreType.DMA,
            pltpu.SemaphoreType.DMA,
            pltpu.SemaphoreType.DMA,
        ],
    )(_sc_body)


_GDN = lax.GatherDimensionNumbers(
    offset_dims=(), collapsed_slice_dims=(0,), start_index_map=(0,))


def _lane_perm(v, idx):
    return lax.gather(v, idx[:, None], _GDN, slice_sizes=(1,),
                      mode=lax.GatherScatterMode.PROMISE_IN_BOUNDS)


def _lane_sum(v):
    lanes = lax.iota(jnp.int32, L)
    for k in (1, 2, 4, 8):
        v = v + _lane_perm(v, lanes ^ k)
    return v


def _sc_body(x_hbm, t2_hbm, w2_hbm, b2_hbm, out_hbm,
             xs_v, idx_v, rows_v, w2_v, b2_v, log_v,
             sem0, sem1, sem2, sem3):
    wid = lax.axis_index("s") * NC + lax.axis_index("c")
    base = wid * BPW

    pltpu.sync_copy(x_hbm.at[pl.ds(base * SEQ, BPW * SEQ)], xs_v)
    pltpu.sync_copy(w2_hbm, w2_v)
    pltpu.sync_copy(b2_hbm, b2_v)

    sems = (sem0, sem1, sem2, sem3)
    offs = [(lax.iota(jnp.int32, L) + k * L) * PROWS for k in range(NPAIR // L)]

    def compute_idx(i, b):
        for q in range(4):
            iq = i + q
            xv = [xs_v[pl.ds(iq * SEQ + k * L, L)] for k in range(SEQ // L)]
            for k in range(NPAIR // L):
                idx_v[b, pl.ds(q * NPAIR + k * L, L)] = (
                    offs[k] + xv[k] * VPAD + xv[k + NPAIR // L])

    def start(b):
        pltpu.async_copy(t2_hbm.at[idx_v.at[b]], rows_v.at[b], sems[b])

    def wait(b):
        pltpu.make_async_copy(t2_hbm.at[idx_v.at[b]], rows_v.at[b],
                              sems[b]).wait()

    lanes = lax.iota(jnp.int32, L)
    bias_c = jnp.float32(NPAIR * QBIAS)

    def one_chunk(q, b, c, dot):
        co = c * L
        s_lo = None
        s_hi = None
        for g in range(4):
            w = rows_v[b, q * NPAIR + 8 * g, pl.ds(co, L)]
            for j in range(1, 8):
                w = w + rows_v[b, q * NPAIR + 8 * g + j, pl.ds(co, L)]
            lo = lax.bitwise_and(w, 65535)
            hi = lax.shift_right_logical(w, 16)
            s_lo = lo if g == 0 else s_lo + lo
            s_hi = hi if g == 0 else s_hi + hi
        hl = jnp.maximum(s_lo.astype(jnp.float32) - bias_c, 0.0)
        hh = jnp.maximum(s_hi.astype(jnp.float32) - bias_c, 0.0)
        return (dot + hl * w2_v[pl.ds(co, L)]
                + hh * w2_v[pl.ds(HIDW + co, L)])

    def accumulate(lane_id, q, b, logacc):
        def chunk_body(c, dot):
            dot = one_chunk(q, b, 2 * c, dot)
            return one_chunk(q, b, 2 * c + 1, dot)

        dot = lax.fori_loop(0, HIDW // (2 * L), chunk_body, b2_v[...])
        tot = _lane_sum(dot)
        return logacc + jnp.where(lanes == lane_id, tot, 0.0)

    compute_idx(0, 0)
    start(0)

    def pair_body(p, logacc):
        i0 = 8 * p
        compute_idx(i0 + 4, 1)
        start(1)
        wait(0)
        for q in range(4):
            logacc = accumulate((i0 + q) & (L - 1), q, 0, logacc)

        @pl.when(p < BPW // 8 - 1)
        def _():
            compute_idx(i0 + 8, 0)
            start(0)

        wait(1)
        for q in range(4):
            logacc = accumulate((i0 + 4 + q) & (L - 1), q, 1, logacc)

        group_done = (p & 1) == 1

        @pl.when(group_done)
        def _():
            log_v[pl.ds((p >> 1) * L, L)] = logacc

        return jnp.where(group_done, 0.0, logacc)

    lax.fori_loop(0, BPW // 8, pair_body, jnp.zeros((L,), jnp.float32))

    pltpu.sync_copy(log_v, out_hbm.at[pl.ds(base, BPW)])


def kernel(batch_x, emb_table, W1, b1, W2, b2):
    w1r = W1.reshape(SEQ, EMB, HID)
    emb_pad = jnp.zeros((VPAD, EMB), jnp.float32).at[:VOCAB].set(emb_table)
    m, mx = _build_m(emb_pad, w1r, b1)

    delta = jnp.maximum(jnp.max(mx), 1e-30) / 1023.0
    table = _build_table(m, 1.0 / delta).reshape(NROWS, HIDW)

    x_flat = batch_x.astype(jnp.int32).reshape(-1)
    w2 = W2.reshape(HID) * delta
    b2_pad = jnp.zeros((L,), jnp.float32).at[0].set(b2[0])
    return _make_sc_forward()(x_flat, table, w2, b2_pad)
